# Initial kernel scaffold; baseline (speedup 1.0000x reference)
import jax
import jax.numpy as jnp
from jax.experimental import pallas as pl


def kernel(x, batch, g_feat, label_emb, deg_emb, gproj_w, gproj_b, w1, b1, w2, b2):
    raise NotImplementedError("write your pallas kernel here")


# speedup vs baseline: 11.6117x; 11.6117x over previous
import jax
import jax.numpy as jnp
from jax import lax
from jax.experimental import pallas as pl

_HIDDEN = 128
_NG = 16
_LV = 102
_DV = 11
_NN = 32768
_COLS = 16
_ROWS = _NN // _COLS
_NREL = 18


def _tc_body(lab_ref, deg_ref, bat_ref, gf_ref, le_ref, de_ref, gw_ref,
             gb_ref, w1a_ref, w1b_ref, b1_ref, w2_ref, b2_ref, out_ref):
    lab = jnp.clip(lab_ref[...], 0, _LV - 1)
    deg = jnp.clip(deg_ref[...], 0, _DV - 1)
    bat = bat_ref[...]
    iota128 = lax.broadcasted_iota(jnp.int32, (_ROWS, 128), 1)
    iota16 = lax.broadcasted_iota(jnp.int32, (_ROWS, _NG), 1)
    acc = jnp.zeros((_NG, 256), jnp.float32)
    for j in range(_COLS):
        onehot_lab = (jnp.broadcast_to(lab[:, j:j + 1], (_ROWS, 128)) == iota128)
        onehot_deg = (jnp.broadcast_to(deg[:, j:j + 1], (_ROWS, 128)) == iota128)
        rhs = jnp.concatenate([onehot_lab, onehot_deg], axis=1).astype(jnp.bfloat16)
        onehot_g = (jnp.broadcast_to(bat[:, j:j + 1], (_ROWS, _NG)) == iota16
                    ).astype(jnp.bfloat16)
        acc += lax.dot_general(onehot_g, rhs, (((0,), (0,)), ((), ())),
                               preferred_element_type=jnp.float32)
    counts_lab = acc[:, :128]
    counts_deg = acc[:, 128:]
    hg = (jnp.dot(counts_lab, le_ref[...], preferred_element_type=jnp.float32, precision=lax.Precision.HIGHEST)
          + jnp.dot(counts_deg, de_ref[...], preferred_element_type=jnp.float32, precision=lax.Precision.HIGHEST))
    gp = jnp.dot(gf_ref[...], gw_ref[...],
                 preferred_element_type=jnp.float32, precision=lax.Precision.HIGHEST) + gb_ref[...]
    hidden = jnp.maximum(
        jnp.dot(hg, w1a_ref[...], preferred_element_type=jnp.float32, precision=lax.Precision.HIGHEST)
        + jnp.dot(gp, w1b_ref[...], preferred_element_type=jnp.float32, precision=lax.Precision.HIGHEST)
        + b1_ref[...], 0.0)
    out_ref[...] = jnp.dot(hidden, w2_ref[...],
                           preferred_element_type=jnp.float32, precision=lax.Precision.HIGHEST) + b2_ref[...]


def kernel(x, batch, g_feat, label_emb, deg_emb, gproj_w, gproj_b,
           w1, b1, w2, b2):
    xi = x.astype(jnp.int32)
    lab = xi[:, 0].reshape(_ROWS, _COLS)
    deg = xi[:, 1].reshape(_ROWS, _COLS)
    bat = batch.astype(jnp.int32).reshape(_ROWS, _COLS)
    gf = jnp.pad(g_feat, ((0, 0), (0, 128 - _LV)))
    le = jnp.pad(label_emb, ((0, 128 - _LV), (0, 0)))
    de = jnp.pad(deg_emb, ((0, 128 - _DV), (0, 0)))
    gw = jnp.pad(gproj_w, ((0, 128 - _LV), (0, 0)))
    gb = gproj_b.reshape(1, _HIDDEN)
    w1a = w1[:_HIDDEN]
    w1b = w1[_HIDDEN:]
    b1r = b1.reshape(1, _HIDDEN)
    w2p = jnp.pad(w2, ((0, 0), (0, 128 - _NREL)))
    b2p = jnp.pad(b2, (0, 128 - _NREL)).reshape(1, 128)
    out = pl.pallas_call(
        _tc_body,
        out_shape=jax.ShapeDtypeStruct((_NG, 128), jnp.float32),
    )(lab, deg, bat, gf, le, de, gw, gb, w1a, w1b, b1r, w2p, b2p)
    return out[:, :_NREL]


# speedup vs baseline: 12.4016x; 1.0680x over previous
import jax
import jax.numpy as jnp
from jax import lax
from jax.experimental import pallas as pl
from jax.experimental.pallas import tpu as pltpu
from jax.experimental.pallas import tpu_sc as plsc

_HIDDEN = 128
_NG = 16
_LV = 102
_DV = 11
_NN = 32768
_NREL = 18

_NTILES = 32
_NPT = _NN // _NTILES
_NBINS = 4096


def _sc_hist_body(lab_hbm, deg_hbm, bat_hbm, cnt_hbm,
                  lab_v, deg_v, bat_v, tab_v):
    cid = lax.axis_index("c")
    sid = lax.axis_index("s")
    wid = cid * 16 + sid
    base = wid * _NPT
    for k in range(_NBINS // 16):
        tab_v[pl.ds(k * 16, 16)] = jnp.zeros((16,), jnp.float32)
    pltpu.sync_copy(lab_hbm.at[pl.ds(base, _NPT)], lab_v)
    pltpu.sync_copy(deg_hbm.at[pl.ds(base, _NPT)], deg_v)
    pltpu.sync_copy(bat_hbm.at[pl.ds(base, _NPT)], bat_v)
    ones = jnp.ones((16,), jnp.float32)
    for i in range(_NPT // 16):
        b16 = bat_v[pl.ds(i * 16, 16)] * 128
        l16 = jnp.clip(lab_v[pl.ds(i * 16, 16)], 0, _LV - 1)
        d16 = jnp.clip(deg_v[pl.ds(i * 16, 16)], 0, _DV - 1)
        plsc.addupdate_scatter(tab_v, [b16 + l16], ones)
        plsc.addupdate_scatter(tab_v, [b16 + d16 + 2048], ones)
    pltpu.sync_copy(tab_v, cnt_hbm.at[wid])


def _sc_hist(lab, deg, bat):
    mesh = plsc.VectorSubcoreMesh(core_axis_name="c", subcore_axis_name="s",
                                  num_cores=2, num_subcores=16)
    return pl.kernel(
        _sc_hist_body,
        out_type=jax.ShapeDtypeStruct((_NTILES, _NBINS), jnp.float32),
        mesh=mesh,
        compiler_params=pltpu.CompilerParams(needs_layout_passes=False),
        scratch_types=[
            pltpu.VMEM((_NPT,), jnp.int32),
            pltpu.VMEM((_NPT,), jnp.int32),
            pltpu.VMEM((_NPT,), jnp.int32),
            pltpu.VMEM((_NBINS,), jnp.float32),
        ],
    )(lab, deg, bat)


def _tc_head_body(cnt_ref, gf_ref, le_ref, de_ref, gw_ref, gb_ref,
                  w1a_ref, w1b_ref, b1_ref, w2_ref, b2_ref, out_ref):
    c = jnp.sum(cnt_ref[...], axis=0)
    counts_lab = c[:_NG]
    counts_deg = c[_NG:]
    hg = (jnp.dot(counts_lab, le_ref[...], preferred_element_type=jnp.float32,
                  precision=lax.Precision.HIGHEST)
          + jnp.dot(counts_deg, de_ref[...], preferred_element_type=jnp.float32,
                    precision=lax.Precision.HIGHEST))
    gp = jnp.dot(gf_ref[...], gw_ref[...], preferred_element_type=jnp.float32,
                 precision=lax.Precision.HIGHEST) + gb_ref[...]
    hidden = jnp.maximum(
        jnp.dot(hg, w1a_ref[...], preferred_element_type=jnp.float32,
                precision=lax.Precision.HIGHEST)
        + jnp.dot(gp, w1b_ref[...], preferred_element_type=jnp.float32,
                  precision=lax.Precision.HIGHEST)
        + b1_ref[...], 0.0)
    out_ref[...] = jnp.dot(hidden, w2_ref[...], preferred_element_type=jnp.float32,
                           precision=lax.Precision.HIGHEST) + b2_ref[...]


def kernel(x, batch, g_feat, label_emb, deg_emb, gproj_w, gproj_b,
           w1, b1, w2, b2):
    xi = x.astype(jnp.int32)
    lab = xi[:, 0]
    deg = xi[:, 1]
    bat = batch.astype(jnp.int32)
    counts = _sc_hist(lab, deg, bat).reshape(_NTILES, 32, 128)
    gf = jnp.pad(g_feat, ((0, 0), (0, 128 - _LV)))
    le = jnp.pad(label_emb, ((0, 128 - _LV), (0, 0)))
    de = jnp.pad(deg_emb, ((0, 128 - _DV), (0, 0)))
    gw = jnp.pad(gproj_w, ((0, 128 - _LV), (0, 0)))
    gb = gproj_b.reshape(1, _HIDDEN)
    w1a = w1[:_HIDDEN]
    w1b = w1[_HIDDEN:]
    b1r = b1.reshape(1, _HIDDEN)
    w2p = jnp.pad(w2, ((0, 0), (0, 128 - _NREL)))
    b2p = jnp.pad(b2, (0, 128 - _NREL)).reshape(1, 128)
    out = pl.pallas_call(
        _tc_head_body,
        out_shape=jax.ShapeDtypeStruct((_NG, 128), jnp.float32),
    )(counts, gf, le, de, gw, gb, w1a, w1b, b1r, w2p, b2p)
    return out[:, :_NREL]
